# baseline (device time: 135223 ns/iter reference)
import jax
import jax.numpy as jnp
from jax import lax
from jax.experimental import pallas as pl
from jax.experimental.pallas import tpu as pltpu

N_DEV = 8


def kernel(x, Wg, Wu, Wd):
    m, k = x.shape
    _, h_per = Wg.shape
    chunk = m // N_DEV
    hblk = 512

    def body(x_ref, wg_ref, wu_ref, wd_ref, out_ref, h_ref, comm_ref,
             rs_send, rs_recv, ag_send, ag_recv):
        my = lax.axis_index("i")
        left = lax.rem(my - 1 + N_DEV, N_DEV)
        right = lax.rem(my + 1, N_DEV)

        barrier = pltpu.get_barrier_semaphore()
        for nbr in (left, right):
            pl.semaphore_signal(
                barrier, inc=1,
                device_id=(nbr,), device_id_type=pl.DeviceIdType.MESH,
            )
        pl.semaphore_wait(barrier, 2)

        xb = x_ref[...].astype(jnp.bfloat16)
        for j in range(0, h_per, hblk):
            wg_b = wg_ref[:, j:j + hblk].astype(jnp.bfloat16)
            wu_b = wu_ref[:, j:j + hblk].astype(jnp.bfloat16)
            gate = jnp.dot(xb, wg_b, preferred_element_type=jnp.float32)
            up = jnp.dot(xb, wu_b, preferred_element_type=jnp.float32)
            h_ref[:, j:j + hblk] = (
                gate * (up * jax.nn.sigmoid(up))
            ).astype(jnp.bfloat16)
        out_ref[...] = jnp.dot(
            h_ref[...], wd_ref[...].astype(jnp.bfloat16),
            preferred_element_type=jnp.float32,
        )

        comm_ref[0, :, :] = out_ref[pl.ds(my * chunk, chunk), :]
        for s in range(N_DEV - 1):
            rdma = pltpu.make_async_remote_copy(
                src_ref=comm_ref.at[s],
                dst_ref=comm_ref.at[s + 1],
                send_sem=rs_send.at[s],
                recv_sem=rs_recv.at[s],
                device_id=(right,),
                device_id_type=pl.DeviceIdType.MESH,
            )
            rdma.start()
            rdma.wait()
            c = lax.rem(my - s - 1 + 2 * N_DEV, N_DEV)
            acc = comm_ref[s + 1, :, :] + out_ref[pl.ds(c * chunk, chunk), :]
            if s < N_DEV - 2:
                comm_ref[s + 1, :, :] = acc
            else:
                out_ref[pl.ds(c * chunk, chunk), :] = acc

        for g in range(N_DEV - 1):
            src_c = lax.rem(my + 1 - g + 2 * N_DEV, N_DEV)
            rdma = pltpu.make_async_remote_copy(
                src_ref=out_ref.at[pl.ds(src_c * chunk, chunk)],
                dst_ref=out_ref.at[pl.ds(src_c * chunk, chunk)],
                send_sem=ag_send.at[g],
                recv_sem=ag_recv.at[g],
                device_id=(right,),
                device_id_type=pl.DeviceIdType.MESH,
            )
            rdma.start()
            rdma.wait()

    return pl.pallas_call(
        body,
        out_shape=jax.ShapeDtypeStruct((m, k), jnp.float32),
        in_specs=[pl.BlockSpec(memory_space=pltpu.VMEM)] * 4,
        out_specs=pl.BlockSpec(memory_space=pltpu.VMEM),
        scratch_shapes=[
            pltpu.VMEM((m, h_per), jnp.bfloat16),
            pltpu.VMEM((N_DEV, chunk, k), jnp.float32),
            pltpu.SemaphoreType.DMA((N_DEV - 1,)),
            pltpu.SemaphoreType.DMA((N_DEV - 1,)),
            pltpu.SemaphoreType.DMA((N_DEV - 1,)),
            pltpu.SemaphoreType.DMA((N_DEV - 1,)),
        ],
        compiler_params=pltpu.CompilerParams(collective_id=0),
    )(x, Wg, Wu, Wd)


# device time: 61171 ns/iter; 2.2106x vs baseline; 2.2106x over previous
import jax
import jax.numpy as jnp
from jax import lax
from jax.experimental import pallas as pl
from jax.experimental.pallas import tpu as pltpu

N_DEV = 8


def kernel(x, Wg, Wu, Wd):
    m, k = x.shape
    _, h_per = Wg.shape
    chunk = m // N_DEV
    hblk = 512

    def body(x_ref, wg_ref, wu_ref, wd_ref, out_ref, h_ref, wdb_ref,
             sbuf, comm, agsrc, agbuf, rs_send, rs_recv, ag_send, ag_recv):
        my = lax.axis_index("i")

        barrier = pltpu.get_barrier_semaphore()
        for p in range(1, N_DEV):
            pl.semaphore_signal(
                barrier, inc=1,
                device_id=(lax.rem(my + p, N_DEV),),
                device_id_type=pl.DeviceIdType.MESH,
            )
        pl.semaphore_wait(barrier, N_DEV - 1)

        xb = x_ref[...].astype(jnp.bfloat16)
        wdb_ref[...] = wd_ref[...].astype(jnp.bfloat16)
        for j in range(0, h_per, hblk):
            wg_b = wg_ref[:, j:j + hblk].astype(jnp.bfloat16)
            wu_b = wu_ref[:, j:j + hblk].astype(jnp.bfloat16)
            gate = jnp.dot(xb, wg_b, preferred_element_type=jnp.float32)
            up = jnp.dot(xb, wu_b, preferred_element_type=jnp.float32)
            h_ref[:, j:j + hblk] = (
                gate * (up * jax.nn.sigmoid(up))
            ).astype(jnp.bfloat16)

        def desc(d, src, dst, ssem, rsem):
            return pltpu.make_async_remote_copy(
                src_ref=src, dst_ref=dst,
                send_sem=ssem.at[d], recv_sem=rsem.at[d],
                device_id=(lax.rem(my + d, N_DEV),),
                device_id_type=pl.DeviceIdType.MESH,
            )

        rs = [desc(d, sbuf.at[d], comm.at[d], rs_send, rs_recv)
              for d in range(1, N_DEV)]
        ag = [desc(d, agsrc, agbuf.at[d], ag_send, ag_recv)
              for d in range(1, N_DEV)]

        own = None
        for d in list(range(1, N_DEV)) + [0]:
            c = lax.rem(my + d, N_DEV)
            p = jnp.dot(
                h_ref[pl.ds(c * chunk, chunk), :], wdb_ref[...],
                preferred_element_type=jnp.float32,
            )
            if d == 0:
                own = p
            else:
                sbuf[d, :, :] = p.astype(jnp.bfloat16)
                rs[d - 1].start()

        for d in range(1, N_DEV):
            rs[d - 1].wait_recv()
        red = own
        for d in range(1, N_DEV):
            red = red + comm[d, :, :].astype(jnp.float32)
        out_ref[pl.ds(my * chunk, chunk), :] = red

        agsrc[...] = red.astype(jnp.bfloat16)
        for d in range(1, N_DEV):
            ag[d - 1].start()
        for d in range(1, N_DEV):
            ag[d - 1].wait_recv()
            c = lax.rem(my - d + N_DEV, N_DEV)
            out_ref[pl.ds(c * chunk, chunk), :] = agbuf[d, :, :].astype(
                jnp.float32
            )

        for d in range(1, N_DEV):
            rs[d - 1].wait_send()
            ag[d - 1].wait_send()

    return pl.pallas_call(
        body,
        out_shape=jax.ShapeDtypeStruct((m, k), jnp.float32),
        in_specs=[pl.BlockSpec(memory_space=pltpu.VMEM)] * 4,
        out_specs=pl.BlockSpec(memory_space=pltpu.VMEM),
        scratch_shapes=[
            pltpu.VMEM((m, h_per), jnp.bfloat16),
            pltpu.VMEM((h_per, k), jnp.bfloat16),
            pltpu.VMEM((N_DEV, chunk, k), jnp.bfloat16),
            pltpu.VMEM((N_DEV, chunk, k), jnp.bfloat16),
            pltpu.VMEM((chunk, k), jnp.bfloat16),
            pltpu.VMEM((N_DEV, chunk, k), jnp.bfloat16),
            pltpu.SemaphoreType.DMA((N_DEV,)),
            pltpu.SemaphoreType.DMA((N_DEV,)),
            pltpu.SemaphoreType.DMA((N_DEV,)),
            pltpu.SemaphoreType.DMA((N_DEV,)),
        ],
        compiler_params=pltpu.CompilerParams(collective_id=0),
    )(x, Wg, Wu, Wd)


# device time: 53244 ns/iter; 2.5397x vs baseline; 1.1489x over previous
import jax
import jax.numpy as jnp
from jax import lax
from jax.experimental import pallas as pl
from jax.experimental.pallas import tpu as pltpu

N_DEV = 8


def kernel(x, Wg, Wu, Wd):
    m, k = x.shape
    _, h_per = Wg.shape
    chunk = m // N_DEV

    def body(x_ref, wg_ref, wu_ref, wd_ref, out_ref,
             sbuf, comm, agsrc, agbuf,
             rs_send, rs_recv, ag_send, ag_recv):
        my = lax.axis_index("i")

        barrier = pltpu.get_barrier_semaphore()
        for p in range(1, N_DEV):
            pl.semaphore_signal(
                barrier, inc=1,
                device_id=(lax.rem(my + p, N_DEV),),
                device_id_type=pl.DeviceIdType.MESH,
            )
        pl.semaphore_wait(barrier, N_DEV - 1)

        def rs_desc(d):
            return pltpu.make_async_remote_copy(
                src_ref=sbuf.at[d], dst_ref=comm.at[d],
                send_sem=rs_send.at[d], recv_sem=rs_recv.at[d],
                device_id=(lax.rem(my + d, N_DEV),),
                device_id_type=pl.DeviceIdType.MESH,
            )

        def ag_desc(d):
            return pltpu.make_async_remote_copy(
                src_ref=agsrc, dst_ref=agbuf.at[d],
                send_sem=ag_send.at[d], recv_sem=ag_recv.at[d],
                device_id=(lax.rem(my + d, N_DEV),),
                device_id_type=pl.DeviceIdType.MESH,
            )

        def mlp_chunk(c):
            xb_c = x_ref[pl.ds(c * chunk, chunk), :]
            gate = jnp.dot(xb_c, wg_ref[...],
                           preferred_element_type=jnp.float32)
            up = jnp.dot(xb_c, wu_ref[...],
                         preferred_element_type=jnp.float32)
            hh = (gate * (up * jax.nn.sigmoid(up))).astype(jnp.bfloat16)
            return jnp.dot(hh, wd_ref[...],
                           preferred_element_type=jnp.float32)

        def send_step(d, carry):
            sbuf[d, :, :] = mlp_chunk(lax.rem(my + d, N_DEV)).astype(
                jnp.bfloat16
            )
            rs_desc(d).start()
            return carry

        lax.fori_loop(1, N_DEV, send_step, 0)
        own = mlp_chunk(my)

        def red_step(d, red):
            rs_desc(d).wait_recv()
            return red + comm[d, :, :].astype(jnp.float32)

        red = lax.fori_loop(1, N_DEV, red_step, own)
        out_ref[pl.ds(my * chunk, chunk), :] = red

        agsrc[...] = red.astype(jnp.bfloat16)

        def ag_start(d, carry):
            ag_desc(d).start()
            return carry

        lax.fori_loop(1, N_DEV, ag_start, 0)

        def ag_wait(d, carry):
            ag_desc(d).wait_recv()
            c = lax.rem(my - d + N_DEV, N_DEV)
            out_ref[pl.ds(c * chunk, chunk), :] = agbuf[d, :, :].astype(
                jnp.float32
            )
            return carry

        lax.fori_loop(1, N_DEV, ag_wait, 0)

        def drain(d, carry):
            rs_desc(d).wait_send()
            ag_desc(d).wait_send()
            return carry

        lax.fori_loop(1, N_DEV, drain, 0)

    call = pl.pallas_call(
        body,
        out_shape=jax.ShapeDtypeStruct((m, k), jnp.float32),
        in_specs=[pl.BlockSpec(memory_space=pltpu.VMEM)] * 4,
        out_specs=pl.BlockSpec(memory_space=pltpu.VMEM),
        scratch_shapes=[
            pltpu.VMEM((N_DEV, chunk, k), jnp.bfloat16),
            pltpu.VMEM((N_DEV, chunk, k), jnp.bfloat16),
            pltpu.VMEM((chunk, k), jnp.bfloat16),
            pltpu.VMEM((N_DEV, chunk, k), jnp.bfloat16),
            pltpu.SemaphoreType.DMA((N_DEV,)),
            pltpu.SemaphoreType.DMA((N_DEV,)),
            pltpu.SemaphoreType.DMA((N_DEV,)),
            pltpu.SemaphoreType.DMA((N_DEV,)),
        ],
        compiler_params=pltpu.CompilerParams(collective_id=0),
    )
    return call(
        x.astype(jnp.bfloat16),
        Wg.astype(jnp.bfloat16),
        Wu.astype(jnp.bfloat16),
        Wd.astype(jnp.bfloat16),
    )
